# Initial kernel scaffold; baseline (speedup 1.0000x reference)
import jax
import jax.numpy as jnp
from jax.experimental import pallas as pl


def kernel(tests, questions, tags, test_types, question_num, W_test, W_question, W_tag, W_test_type, W_pos, W_qnum):
    raise NotImplementedError("write your pallas kernel here")


# speedup vs baseline: 4.4269x; 4.4269x over previous
import functools

import jax
import jax.numpy as jnp
from jax import lax
from jax.experimental import pallas as pl
from jax.experimental.pallas import tpu as pltpu
from jax.experimental.pallas import tpu_sc as plsc

B, S, D = 4096, 200, 64
NW = 32
ROWS_PER_W = B // NW
L = 16
C0, C1 = 128, S - 128
NT = 5


def _body(tests, questions, tags, ttypes, qnums,
          w_test, w_quest, w_tag, w_ttype, w_pos, w_qnum,
          out, bufs, outbuf, posv, idx_a, idx_b, sem_i, sem_g):
    wid = lax.axis_index("s") * 2 + lax.axis_index("c")

    pltpu.sync_copy(w_pos, posv)

    idx_hbms = (tests, questions, tags, ttypes, qnums)
    w_hbms = (w_test, w_quest, w_tag, w_ttype, w_qnum)

    def row_body(r, carry):
        base = (wid * ROWS_PER_W + r) * S

        waits = []
        for i in range(NT):
            waits.append(pltpu.async_copy(
                idx_hbms[i].at[pl.ds(base, C0)], idx_a.at[i], sem_i))
            waits.append(pltpu.async_copy(
                idx_hbms[i].at[pl.ds(base + C0, C1)],
                idx_b.at[i, pl.ds(0, C1)], sem_i))
        for w in waits:
            w.wait()

        waits = []
        for i in range(NT):
            waits.append(pltpu.async_copy(
                w_hbms[i].at[idx_a.at[i]], bufs.at[i, pl.ds(0, C0)], sem_g))
            waits.append(pltpu.async_copy(
                w_hbms[i].at[idx_b.at[i, pl.ds(0, C1)]],
                bufs.at[i, pl.ds(C0, C1)], sem_g))
        for w in waits:
            w.wait()

        def tok_body(t, carry2):
            for c in range(D // L):
                ds = pl.ds(c * L, L)
                v = posv[t, ds]
                for i in range(NT):
                    v = v + bufs[i, t, ds]
                outbuf[t, ds] = v
            return carry2

        lax.fori_loop(0, S, tok_body, 0, unroll=2)

        pltpu.sync_copy(outbuf, out.at[pl.ds(base, S)])
        return carry

    lax.fori_loop(0, ROWS_PER_W, row_body, 0)


@functools.partial(jax.jit, static_argnames=())
def _run(tests, questions, tags, ttypes, qnums,
         w_test, w_quest, w_tag, w_ttype, w_pos, w_qnum):
    mesh = plsc.VectorSubcoreMesh(core_axis_name="c", subcore_axis_name="s")
    fn = pl.kernel(
        _body,
        out_type=jax.ShapeDtypeStruct((B * S, D), jnp.float32),
        mesh=mesh,
        compiler_params=pltpu.CompilerParams(use_tc_tiling_on_sc=False),
        scratch_types=[
            pltpu.VMEM((NT, S, D), jnp.float32),
            pltpu.VMEM((S, D), jnp.float32),
            pltpu.VMEM((S, D), jnp.float32),
            pltpu.VMEM((NT, C0), jnp.int32),
            pltpu.VMEM((NT, C0), jnp.int32),
            pltpu.SemaphoreType.DMA,
            pltpu.SemaphoreType.DMA,
        ],
    )
    return fn(tests, questions, tags, ttypes, qnums,
              w_test, w_quest, w_tag, w_ttype, w_pos, w_qnum)


def kernel(tests, questions, tags, test_types, question_num,
           W_test, W_question, W_tag, W_test_type, W_pos, W_qnum):
    flat = lambda x: x.reshape(-1).astype(jnp.int32)
    out = _run(flat(tests), flat(questions), flat(tags), flat(test_types),
               flat(question_num),
               W_test, W_question, W_tag, W_test_type, W_pos, W_qnum)
    return out.reshape(B, S, D)
